# baseline (speedup 1.0000x reference)
import functools

import jax
import jax.numpy as jnp
from jax import lax
from jax.experimental import pallas as pl
from jax.experimental.pallas import tpu as pltpu
from jax.experimental.pallas import tpu_sc as plsc

R, N = 128, 32768
NC, NS, L = 2, 16, 16
NW = NC * NS
ROWS_PER_W = R // NW
CHUNK = 8192
NCHUNK = N // CHUNK
K = 8
ITERS = CHUNK // (L * K)
BIG = 2**31 - 1


def _merge(a, b):
    av, ai = a
    bv, bi = b
    take_b = (bv < av) | ((bv == av) & (bi < ai))
    return jnp.where(take_b, bv, av), jnp.where(take_b, bi, ai)


def _body(x_hbm, out_hbm, buf0, buf1, res_v, sem0, sem1):
    wid = lax.axis_index("s") * NC + lax.axis_index("c")
    row0 = wid * ROWS_PER_W
    bufs = (buf0, buf1)
    sems = (sem0, sem1)
    lane = lax.iota(jnp.int32, L)

    def start(t):
        r, c = divmod(t, NCHUNK)
        off = pl.multiple_of((row0 + r) * N + c * CHUNK, CHUNK)
        b = t % 2
        return pltpu.async_copy(x_hbm.at[pl.ds(off, CHUNK)], bufs[b], sems[b])

    total = ROWS_PER_W * NCHUNK
    pending = {0: start(0)}
    res = jnp.zeros((L,), jnp.int32)

    for r in range(ROWS_PER_W):
        accs = [(jnp.full((L,), jnp.inf, jnp.float32), jnp.zeros((L,), jnp.int32))
                for _ in range(K)]
        for c in range(NCHUNK):
            t = r * NCHUNK + c
            pending.pop(t).wait()
            if t + 1 < total:
                pending[t + 1] = start(t + 1)
            buf = bufs[t % 2]

            def step(i, carry, buf=buf):
                ivec, flat = carry
                new = []
                for k in range(K):
                    bv, bi = flat[2 * k], flat[2 * k + 1]
                    v = buf[pl.ds(i * (L * K) + k * L, L)]
                    m = v < bv
                    new.append(jnp.where(m, v, bv))
                    new.append(jnp.where(m, ivec, bi))
                return ivec + 1, tuple(new)

            flat = tuple(x for acc in accs for x in acc)
            ivec0 = jnp.full((L,), c * ITERS, jnp.int32)
            _, flat = lax.fori_loop(0, ITERS, step, (ivec0, flat))
            accs = [(flat[2 * k], flat[2 * k + 1]) for k in range(K)]

        full = [(bv, (bi * K + k) * L + lane) for k, (bv, bi) in enumerate(accs)]
        best = full[0]
        for other in full[1:]:
            best = _merge(best, other)
        bv, bi = best
        for off in (1, 2, 4, 8):
            perm = (lane + off) % L
            ov = bv.at[perm].get(mode="promise_in_bounds")
            oi = bi.at[perm].get(mode="promise_in_bounds")
            bv, bi = _merge((bv, bi), (ov, oi))
        res = jnp.where(lane == r, bi, res)

    res_v[...] = res
    pltpu.sync_copy(res_v, out_hbm.at[wid])


@functools.partial(jax.jit, static_argnums=())
def kernel(x):
    mesh = plsc.VectorSubcoreMesh(core_axis_name="c", subcore_axis_name="s",
                                  num_cores=NC, num_subcores=NS)
    run = pl.kernel(
        _body,
        out_type=jax.ShapeDtypeStruct((NW, L), jnp.int32),
        mesh=mesh,
        scratch_types=[
            pltpu.VMEM((CHUNK,), jnp.float32),
            pltpu.VMEM((CHUNK,), jnp.float32),
            pltpu.VMEM((L,), jnp.int32),
            pltpu.SemaphoreType.DMA,
            pltpu.SemaphoreType.DMA,
        ],
    )
    out2d = run(x.reshape(-1))
    return out2d[:, :ROWS_PER_W].reshape(R).astype(jnp.int64)
